# baseline (device time: 40269 ns/iter reference)
import functools
import os

import jax
import jax.numpy as jnp
from jax import lax
from jax.experimental import pallas as pl
from jax.experimental.pallas import tpu as pltpu

_MODE = os.environ.get("SPARSE_KERNEL_MODE", "")
_DO_COMM = _MODE != "compute_only"
_DO_COMPUTE = _MODE != "comm_only"

N_DEV = 4
B = 2
S = 512
HQ = 8
DH = 64
HD = HQ * DH
E = 768
WINDOW = 128
NGLOB = 32
SCALE = 0.125


def _fused(x, Wq, k2, v2, Wo):

    def body(x_ref, wq_ref, k2_ref, v2_ref, wo_ref, out_ref,
             kown, vown, q_ref, acc_ref, l_ref,
             hlk, hlv, hrk, hrv, gk, gv, gq,
             pacc_s, pl_s, pacc_r, pl_r,
             send_sems, recv_sems):
        my = lax.axis_index("i")
        left = lax.rem(my + N_DEV - 1, N_DEV)
        right = lax.rem(my + 1, N_DEV)

        barrier_sem = pltpu.get_barrier_semaphore()
        for d in range(N_DEV):
            pl.semaphore_signal(
                barrier_sem, inc=1,
                device_id=(jnp.int32(d),),
                device_id_type=pl.DeviceIdType.MESH,
            )
        pl.semaphore_wait(barrier_sem, N_DEV)

        for b in range(B):
            kown[b] = k2_ref[b].astype(jnp.bfloat16)
            vown[b] = v2_ref[b].astype(jnp.bfloat16)

        def rdma(src, dst, si, ri, dev):
            return pltpu.make_async_remote_copy(
                src_ref=src, dst_ref=dst,
                send_sem=send_sems.at[si], recv_sem=recv_sems.at[ri],
                device_id=(dev,), device_id_type=pl.DeviceIdType.MESH,
            )

        hkr = rdma(kown.at[:, pl.ds(S - WINDOW, WINDOW), :], hlk, 0, 0, right)
        hvr = rdma(vown.at[:, pl.ds(S - WINDOW, WINDOW), :], hlv, 1, 1, right)
        hkl = rdma(kown.at[:, pl.ds(0, WINDOW), :], hrk, 2, 2, left)
        hvl = rdma(vown.at[:, pl.ds(0, WINDOW), :], hrv, 3, 3, left)
        if _DO_COMM:
            hkr.start()
            hvr.start()
            hkl.start()
            hvl.start()

            @pl.when(my == 0)
            def _():
                for j, t in enumerate((1, 2, 3)):
                    rdma(kown.at[:, pl.ds(0, NGLOB), :], gk,
                         4 + 3 * j, 4, t).start()
                    rdma(vown.at[:, pl.ds(0, NGLOB), :], gv,
                         5 + 3 * j, 5, t).start()

        wq = wq_ref[...].astype(jnp.bfloat16)
        for b in range(B):
            q_ref[b] = jnp.dot(
                x_ref[b].astype(jnp.bfloat16), wq,
                preferred_element_type=jnp.float32,
            ).astype(jnp.bfloat16)

        if _DO_COMM:
            @pl.when(my == 0)
            def _():
                for j, t in enumerate((1, 2, 3)):
                    rdma(q_ref.at[:, pl.ds(0, NGLOB), :], gq,
                         6 + 3 * j, 6, t).start()

        def process(kref, vref, ncols, cbase, rects, mask_mode, init):
            del ncols
            for r0, nr, c0, nc in rects:
                qi = my * S + r0 + lax.broadcasted_iota(
                    jnp.int32, (nr, nc), 0)
                ki = cbase + c0 + lax.broadcasted_iota(
                    jnp.int32, (nr, nc), 1)
                if mask_mode == "full":
                    mask = ((jnp.abs(qi - ki) <= WINDOW)
                            | (ki < NGLOB) | (qi < NGLOB))
                elif mask_mode == "window32":
                    mask = (jnp.abs(qi - ki) <= WINDOW) & (qi >= NGLOB)
                else:
                    mask = None
                bias = (None if mask is None
                        else jnp.where(mask, 0.0, -1e9).astype(jnp.float32))
                for b in range(B):
                    kb = kref[b, c0:c0 + nc, :]
                    vb = vref[b, c0:c0 + nc, :]
                    for h in range(HQ):
                        sl = slice(h * DH, (h + 1) * DH)
                        s = lax.dot_general(
                            q_ref[b, r0:r0 + nr, sl], kb[:, sl],
                            (((1,), (1,)), ((), ())),
                            preferred_element_type=jnp.float32,
                        ) * SCALE
                        if bias is not None:
                            s = s + bias
                        w = jnp.exp(s)
                        lsum = jnp.sum(w, axis=1, keepdims=True)
                        pv = jnp.dot(w.astype(jnp.bfloat16), vb[:, sl],
                                     preferred_element_type=jnp.float32)
                        if init:
                            l_ref[b, r0:r0 + nr, h:h + 1] = lsum
                            acc_ref[b, r0:r0 + nr, sl] = pv
                        else:
                            l_ref[b, r0:r0 + nr, h:h + 1] += lsum
                            acc_ref[b, r0:r0 + nr, sl] += pv

        if _DO_COMPUTE:
            process(kown, vown, S, my * S,
                    [(0, 256, 0, 384), (256, 256, 128, 384)], "full", True)
            process(kown, vown, S, my * S,
                    [(256, 256, 0, NGLOB), (0, NGLOB, 384, WINDOW)],
                    "full", False)

        grecv_k = rdma(kown.at[:, pl.ds(0, NGLOB), :], gk, 4, 4, 0)
        grecv_v = rdma(vown.at[:, pl.ds(0, NGLOB), :], gv, 4, 5, 0)
        grecv_q = rdma(q_ref.at[:, pl.ds(0, NGLOB), :], gq, 4, 6, 0)
        psend_a = rdma(pacc_s, pacc_r.at[my - 1], 4, 5 + 2 * my, 0)
        psend_l = rdma(pl_s, pl_r.at[my - 1], 5, 6 + 2 * my, 0)

        @pl.when(my != 0)
        def _():
            if _DO_COMM:
                grecv_q.wait_recv()
            if _DO_COMPUTE:
                for b in range(B):
                    for h in range(HQ):
                        sl = slice(h * DH, (h + 1) * DH)
                        s = lax.dot_general(
                            gq[b, :, sl], kown[b, :, sl],
                            (((1,), (1,)), ((), ())),
                            preferred_element_type=jnp.float32,
                        ) * SCALE
                        w = jnp.exp(s)
                        pl_s[b, :, h:h + 1] = jnp.sum(w, 1, keepdims=True)
                        pacc_s[b, :, sl] = jnp.dot(
                            w.astype(jnp.bfloat16), vown[b, :, sl],
                            preferred_element_type=jnp.float32)
            if _DO_COMM:
                psend_a.start()
                psend_l.start()

        if _DO_COMM:
            hkr.wait_recv()
            hvr.wait_recv()
            hkl.wait_recv()
            hvl.wait_recv()
        if _DO_COMPUTE:
            process(hlk, hlv, WINDOW, left * S + (S - WINDOW),
                    [(0, WINDOW, 0, WINDOW)], "window32", False)
            process(hrk, hrv, WINDOW, right * S,
                    [(S - WINDOW, WINDOW, 0, WINDOW)], "window32", False)

        @pl.when(my != 0)
        def _():
            if _DO_COMM:
                grecv_k.wait_recv()
                grecv_v.wait_recv()
            if _DO_COMPUTE:
                process(gk, gv, NGLOB, 0, [(0, S, 0, NGLOB)], "none", False)

        @pl.when(my == 0)
        def _():
            if _DO_COMM:
                for j in range(3):
                    rdma(pacc_s, pacc_r.at[j], 4, 7 + 2 * j, 0).wait_recv()
                    rdma(pl_s, pl_r.at[j], 5, 8 + 2 * j, 0).wait_recv()
            if _DO_COMPUTE:
                for j in range(3):
                    for b in range(B):
                        acc_ref[b, 0:NGLOB, :] += pacc_r[j, b]
                        l_ref[b, 0:NGLOB, :] += pl_r[j, b]

        wo = wo_ref[...].astype(jnp.bfloat16)
        for b in range(B):
            linv = 1.0 / l_ref[b]
            o = jnp.zeros((S, E), jnp.float32)
            for h in range(HQ):
                sl = slice(h * DH, (h + 1) * DH)
                ctx = acc_ref[b, :, sl] * linv[:, h:h + 1]
                o += jnp.dot(ctx.astype(jnp.bfloat16),
                             wo[sl, :],
                             preferred_element_type=jnp.float32)
            out_ref[b] = o

        if _DO_COMM:
            hkr.wait_send()
            hvr.wait_send()
            hkl.wait_send()
            hvl.wait_send()

            @pl.when(my == 0)
            def _():
                for j in range(3):
                    rdma(kown.at[:, pl.ds(0, NGLOB), :], gk,
                         4 + 3 * j, 4, 1).wait_send()
                    rdma(vown.at[:, pl.ds(0, NGLOB), :], gv,
                         5 + 3 * j, 5, 1).wait_send()
                    rdma(q_ref.at[:, pl.ds(0, NGLOB), :], gq,
                         6 + 3 * j, 6, 1).wait_send()

            @pl.when(my != 0)
            def _():
                psend_a.wait_send()
                psend_l.wait_send()

        @functools.partial(pl.run_scoped,
                           second_barrier=pltpu.SemaphoreType.REGULAR)
        def _(second_barrier):
            for d in range(N_DEV):
                pl.semaphore_signal(
                    second_barrier, inc=1,
                    device_id=(jnp.int32(d),),
                    device_id_type=pl.DeviceIdType.MESH,
                )
            pl.semaphore_wait(second_barrier, N_DEV)

    return pl.pallas_call(
        body,
        out_shape=jax.ShapeDtypeStruct((B, S, E), jnp.float32),
        in_specs=[pl.BlockSpec(memory_space=pltpu.VMEM)] * 5,
        out_specs=pl.BlockSpec(memory_space=pltpu.VMEM),
        scratch_shapes=[
            pltpu.VMEM((B, S, HD), jnp.bfloat16),
            pltpu.VMEM((B, S, HD), jnp.bfloat16),
            pltpu.VMEM((B, S, HD), jnp.bfloat16),
            pltpu.VMEM((B, S, HD), jnp.float32),
            pltpu.VMEM((B, S, HQ), jnp.float32),
            pltpu.VMEM((B, WINDOW, HD), jnp.bfloat16),
            pltpu.VMEM((B, WINDOW, HD), jnp.bfloat16),
            pltpu.VMEM((B, WINDOW, HD), jnp.bfloat16),
            pltpu.VMEM((B, WINDOW, HD), jnp.bfloat16),
            pltpu.VMEM((B, NGLOB, HD), jnp.bfloat16),
            pltpu.VMEM((B, NGLOB, HD), jnp.bfloat16),
            pltpu.VMEM((B, NGLOB, HD), jnp.bfloat16),
            pltpu.VMEM((B, NGLOB, HD), jnp.float32),
            pltpu.VMEM((B, NGLOB, HQ), jnp.float32),
            pltpu.VMEM((3, B, NGLOB, HD), jnp.float32),
            pltpu.VMEM((3, B, NGLOB, HQ), jnp.float32),
            pltpu.SemaphoreType.DMA((13,)),
            pltpu.SemaphoreType.DMA((13,)),
        ],
        compiler_params=pltpu.CompilerParams(collective_id=0),
    )(x, Wq, k2, v2, Wo)


def kernel(x, Wq, K_ext, V_ext, Wo):
    k2 = K_ext.reshape(B, S, HD)
    v2 = V_ext.reshape(B, S, HD)
    return _fused(x, Wq, k2, v2, Wo)


# device time: 35442 ns/iter; 1.1362x vs baseline; 1.1362x over previous
import functools
import os

import jax
import jax.numpy as jnp
from jax import lax
from jax.experimental import pallas as pl
from jax.experimental.pallas import tpu as pltpu

_MODE = os.environ.get("SPARSE_KERNEL_MODE", "")
_DO_COMM = _MODE != "compute_only"
_DO_COMPUTE = _MODE != "comm_only"

N_DEV = 4
B = 2
S = 512
HQ = 8
DH = 64
HD = HQ * DH
E = 768
WINDOW = 128
NGLOB = 32
SCALE = 0.125


def _fused(x, Wq, k2, v2, Wo):

    def body(x_ref, wq_ref, k2_ref, v2_ref, wo_ref, out_ref,
             kown, vown, q_ref, acc_ref, l_ref,
             hlk, hlv, hrk, hrv, gk, gv, gq,
             pacc_s, pl_s, pacc_r, pl_r,
             send_sems, recv_sems):
        my = lax.axis_index("i")
        left = lax.rem(my + N_DEV - 1, N_DEV)
        right = lax.rem(my + 1, N_DEV)

        barrier_sem = pltpu.get_barrier_semaphore()
        for d in range(N_DEV):
            pl.semaphore_signal(
                barrier_sem, inc=1,
                device_id=(jnp.int32(d),),
                device_id_type=pl.DeviceIdType.MESH,
            )
        pl.semaphore_wait(barrier_sem, N_DEV)

        for b in range(B):
            kown[b] = k2_ref[b].astype(jnp.bfloat16)
            vown[b] = v2_ref[b].astype(jnp.bfloat16)

        def rdma(src, dst, si, ri, dev):
            return pltpu.make_async_remote_copy(
                src_ref=src, dst_ref=dst,
                send_sem=send_sems.at[si], recv_sem=recv_sems.at[ri],
                device_id=(dev,), device_id_type=pl.DeviceIdType.MESH,
            )

        hkr = rdma(kown.at[:, pl.ds(S - WINDOW, WINDOW), :], hlk, 0, 0, right)
        hvr = rdma(vown.at[:, pl.ds(S - WINDOW, WINDOW), :], hlv, 1, 1, right)
        hkl = rdma(kown.at[:, pl.ds(0, WINDOW), :], hrk, 2, 2, left)
        hvl = rdma(vown.at[:, pl.ds(0, WINDOW), :], hrv, 3, 3, left)
        if _DO_COMM:
            hkr.start()
            hvr.start()
            hkl.start()
            hvl.start()

            @pl.when(my == 0)
            def _():
                for j, t in enumerate((1, 2, 3)):
                    rdma(kown.at[:, pl.ds(0, NGLOB), :], gk,
                         4 + 3 * j, 4, t).start()
                    rdma(vown.at[:, pl.ds(0, NGLOB), :], gv,
                         5 + 3 * j, 5, t).start()

        wq = (wq_ref[...] * SCALE).astype(jnp.bfloat16)
        for b in range(B):
            q_ref[b] = jnp.dot(
                x_ref[b].astype(jnp.bfloat16), wq,
                preferred_element_type=jnp.float32,
            ).astype(jnp.bfloat16)

        if _DO_COMM:
            @pl.when(my == 0)
            def _():
                for j, t in enumerate((1, 2, 3)):
                    rdma(q_ref.at[:, pl.ds(0, NGLOB), :], gq,
                         6 + 3 * j, 6, t).start()

        def process(kref, vref, ncols, cbase, rects, mask_mode, init):
            del ncols
            for r0, nr, c0, nc in rects:
                qi = my * S + r0 + lax.broadcasted_iota(
                    jnp.int32, (nr, nc), 0)
                ki = cbase + c0 + lax.broadcasted_iota(
                    jnp.int32, (nr, nc), 1)
                if mask_mode == "full":
                    mask = ((jnp.abs(qi - ki) <= WINDOW)
                            | (ki < NGLOB) | (qi < NGLOB))
                elif mask_mode == "window32":
                    mask = (jnp.abs(qi - ki) <= WINDOW) & (qi >= NGLOB)
                else:
                    mask = None
                bias = (None if mask is None
                        else jnp.where(mask, 0.0, -1e9).astype(jnp.float32))
                ones_col = jnp.ones((nc, 1), jnp.bfloat16)
                for b in range(B):
                    kb = kref[b, c0:c0 + nc, :]
                    vb = vref[b, c0:c0 + nc, :]
                    for h in range(HQ):
                        sl = slice(h * DH, (h + 1) * DH)
                        s = lax.dot_general(
                            q_ref[b, r0:r0 + nr, sl], kb[:, sl],
                            (((1,), (1,)), ((), ())),
                            preferred_element_type=jnp.float32,
                        )
                        if bias is not None:
                            s = s + bias
                        w = jnp.exp(s).astype(jnp.bfloat16)
                        lsum = jnp.dot(w, ones_col,
                                       preferred_element_type=jnp.float32)
                        pv = jnp.dot(w, vb[:, sl],
                                     preferred_element_type=jnp.float32)
                        if init:
                            l_ref[b, r0:r0 + nr, h:h + 1] = lsum
                            acc_ref[b, r0:r0 + nr, sl] = pv
                        else:
                            l_ref[b, r0:r0 + nr, h:h + 1] += lsum
                            acc_ref[b, r0:r0 + nr, sl] += pv

        if _DO_COMPUTE:
            process(kown, vown, S, my * S,
                    [(0, 256, 0, 384), (256, 256, 128, 384)], "full", True)
            process(kown, vown, S, my * S,
                    [(256, 256, 0, NGLOB), (0, NGLOB, 384, WINDOW)],
                    "full", False)

        grecv_k = rdma(kown.at[:, pl.ds(0, NGLOB), :], gk, 4, 4, 0)
        grecv_v = rdma(vown.at[:, pl.ds(0, NGLOB), :], gv, 4, 5, 0)
        grecv_q = rdma(q_ref.at[:, pl.ds(0, NGLOB), :], gq, 4, 6, 0)
        psend_a = rdma(pacc_s, pacc_r.at[my - 1], 4, 5 + 2 * my, 0)
        psend_l = rdma(pl_s, pl_r.at[my - 1], 5, 6 + 2 * my, 0)

        @pl.when(my != 0)
        def _():
            if _DO_COMM:
                grecv_q.wait_recv()
            if _DO_COMPUTE:
                ones_col = jnp.ones((S, 1), jnp.bfloat16)
                for b in range(B):
                    for h in range(HQ):
                        sl = slice(h * DH, (h + 1) * DH)
                        s = lax.dot_general(
                            gq[b, :, sl], kown[b, :, sl],
                            (((1,), (1,)), ((), ())),
                            preferred_element_type=jnp.float32,
                        )
                        w = jnp.exp(s).astype(jnp.bfloat16)
                        pl_s[b, :, h:h + 1] = jnp.dot(
                            w, ones_col, preferred_element_type=jnp.float32)
                        pacc_s[b, :, sl] = jnp.dot(
                            w, vown[b, :, sl],
                            preferred_element_type=jnp.float32)
            if _DO_COMM:
                psend_a.start()
                psend_l.start()

        if _DO_COMM:
            hkr.wait_recv()
            hvr.wait_recv()
            hkl.wait_recv()
            hvl.wait_recv()
        if _DO_COMPUTE:
            process(hlk, hlv, WINDOW, left * S + (S - WINDOW),
                    [(0, WINDOW, 0, WINDOW)], "window32", False)
            process(hrk, hrv, WINDOW, right * S,
                    [(S - WINDOW, WINDOW, 0, WINDOW)], "window32", False)

        @pl.when(my != 0)
        def _():
            if _DO_COMM:
                grecv_k.wait_recv()
                grecv_v.wait_recv()
            if _DO_COMPUTE:
                process(gk, gv, NGLOB, 0, [(0, S, 0, NGLOB)], "none", False)

        @pl.when(my == 0)
        def _():
            if _DO_COMM:
                for j in range(3):
                    rdma(pacc_s, pacc_r.at[j], 4, 7 + 2 * j, 0).wait_recv()
                    rdma(pl_s, pl_r.at[j], 5, 8 + 2 * j, 0).wait_recv()
            if _DO_COMPUTE:
                for j in range(3):
                    for b in range(B):
                        acc_ref[b, 0:NGLOB, :] += pacc_r[j, b]
                        l_ref[b, 0:NGLOB, :] += pl_r[j, b]

        if _DO_COMM:
            hkr.wait_send()
            hvr.wait_send()
            hkl.wait_send()
            hvl.wait_send()

            @pl.when(my == 0)
            def _():
                for j in range(3):
                    rdma(kown.at[:, pl.ds(0, NGLOB), :], gk,
                         4 + 3 * j, 4, 1).wait_send()
                    rdma(vown.at[:, pl.ds(0, NGLOB), :], gv,
                         5 + 3 * j, 5, 1).wait_send()
                    rdma(q_ref.at[:, pl.ds(0, NGLOB), :], gq,
                         6 + 3 * j, 6, 1).wait_send()

            @pl.when(my != 0)
            def _():
                psend_a.wait_send()
                psend_l.wait_send()

        wo = wo_ref[...].astype(jnp.bfloat16)
        for b in range(B):
            linv = 1.0 / l_ref[b]
            for h in range(HQ):
                sl = slice(h * DH, (h + 1) * DH)
                q_ref[b, :, sl] = (
                    acc_ref[b, :, sl] * linv[:, h:h + 1]
                ).astype(jnp.bfloat16)
        for b in range(B):
            out_ref[b] = jnp.dot(q_ref[b], wo,
                                 preferred_element_type=jnp.float32)

        @functools.partial(pl.run_scoped,
                           second_barrier=pltpu.SemaphoreType.REGULAR)
        def _(second_barrier):
            for d in range(N_DEV):
                pl.semaphore_signal(
                    second_barrier, inc=1,
                    device_id=(jnp.int32(d),),
                    device_id_type=pl.DeviceIdType.MESH,
                )
            pl.semaphore_wait(second_barrier, N_DEV)

    return pl.pallas_call(
        body,
        out_shape=jax.ShapeDtypeStruct((B, S, E), jnp.float32),
        in_specs=[pl.BlockSpec(memory_space=pltpu.VMEM)] * 5,
        out_specs=pl.BlockSpec(memory_space=pltpu.VMEM),
        scratch_shapes=[
            pltpu.VMEM((B, S, HD), jnp.bfloat16),
            pltpu.VMEM((B, S, HD), jnp.bfloat16),
            pltpu.VMEM((B, S, HD), jnp.bfloat16),
            pltpu.VMEM((B, S, HD), jnp.float32),
            pltpu.VMEM((B, S, HQ), jnp.float32),
            pltpu.VMEM((B, WINDOW, HD), jnp.bfloat16),
            pltpu.VMEM((B, WINDOW, HD), jnp.bfloat16),
            pltpu.VMEM((B, WINDOW, HD), jnp.bfloat16),
            pltpu.VMEM((B, WINDOW, HD), jnp.bfloat16),
            pltpu.VMEM((B, NGLOB, HD), jnp.bfloat16),
            pltpu.VMEM((B, NGLOB, HD), jnp.bfloat16),
            pltpu.VMEM((B, NGLOB, HD), jnp.bfloat16),
            pltpu.VMEM((B, NGLOB, HD), jnp.float32),
            pltpu.VMEM((B, NGLOB, HQ), jnp.float32),
            pltpu.VMEM((3, B, NGLOB, HD), jnp.float32),
            pltpu.VMEM((3, B, NGLOB, HQ), jnp.float32),
            pltpu.SemaphoreType.DMA((13,)),
            pltpu.SemaphoreType.DMA((13,)),
        ],
        compiler_params=pltpu.CompilerParams(collective_id=0),
    )(x, Wq, k2, v2, Wo)


def kernel(x, Wq, K_ext, V_ext, Wo):
    k2 = K_ext.reshape(B, S, HD)
    v2 = V_ext.reshape(B, S, HD)
    return _fused(x, Wq, k2, v2, Wo)


# device time: 35273 ns/iter; 1.1416x vs baseline; 1.0048x over previous
import functools
import os

import jax
import jax.numpy as jnp
from jax import lax
from jax.experimental import pallas as pl
from jax.experimental.pallas import tpu as pltpu

_MODE = os.environ.get("SPARSE_KERNEL_MODE", "")
_DO_COMM = _MODE != "compute_only"
_DO_COMPUTE = _MODE != "comm_only"

N_DEV = 4
B = 2
S = 512
HQ = 8
DH = 64
HD = HQ * DH
E = 768
WINDOW = 128
NGLOB = 32
SCALE = 0.125


def _fused(x, Wq, k2, v2, Wo):

    def body(x_ref, wq_ref, k2_ref, v2_ref, wo_ref, out_ref,
             kf32, vf32, kown, vown, q_ref, acc_ref, l_ref,
             hlk, hlv, hrk, hrv, gk, gv, gq,
             pacc_s, pl_s, pacc_r, pl_r,
             copy_sems, send_sems, recv_sems):
        my = lax.axis_index("i")
        left = lax.rem(my + N_DEV - 1, N_DEV)
        right = lax.rem(my + 1, N_DEV)

        kcopy = pltpu.make_async_copy(k2_ref, kf32, copy_sems.at[0])
        vcopy = pltpu.make_async_copy(v2_ref, vf32, copy_sems.at[1])
        kcopy.start()
        vcopy.start()

        barrier_sem = pltpu.get_barrier_semaphore()
        for d in range(N_DEV):
            pl.semaphore_signal(
                barrier_sem, inc=1,
                device_id=(jnp.int32(d),),
                device_id_type=pl.DeviceIdType.MESH,
            )

        wq = (wq_ref[...] * SCALE).astype(jnp.bfloat16)
        for b in range(B):
            q_ref[b] = jnp.dot(
                x_ref[b].astype(jnp.bfloat16), wq,
                preferred_element_type=jnp.float32,
            ).astype(jnp.bfloat16)

        pl.semaphore_wait(barrier_sem, N_DEV)
        kcopy.wait()
        vcopy.wait()

        for b in range(B):
            kown[b] = kf32[b].astype(jnp.bfloat16)
            vown[b] = vf32[b].astype(jnp.bfloat16)

        def rdma(src, dst, si, ri, dev):
            return pltpu.make_async_remote_copy(
                src_ref=src, dst_ref=dst,
                send_sem=send_sems.at[si], recv_sem=recv_sems.at[ri],
                device_id=(dev,), device_id_type=pl.DeviceIdType.MESH,
            )

        hkr = rdma(kown.at[:, pl.ds(S - WINDOW, WINDOW), :], hlk, 0, 0, right)
        hvr = rdma(vown.at[:, pl.ds(S - WINDOW, WINDOW), :], hlv, 1, 1, right)
        hkl = rdma(kown.at[:, pl.ds(0, WINDOW), :], hrk, 2, 2, left)
        hvl = rdma(vown.at[:, pl.ds(0, WINDOW), :], hrv, 3, 3, left)
        if _DO_COMM:
            hkr.start()
            hvr.start()
            hkl.start()
            hvl.start()

            @pl.when(my == 0)
            def _():
                for j, t in enumerate((1, 2, 3)):
                    rdma(kown.at[:, pl.ds(0, NGLOB), :], gk,
                         4 + 3 * j, 4, t).start()
                    rdma(vown.at[:, pl.ds(0, NGLOB), :], gv,
                         5 + 3 * j, 5, t).start()

        if _DO_COMM:
            @pl.when(my == 0)
            def _():
                for j, t in enumerate((1, 2, 3)):
                    rdma(q_ref.at[:, pl.ds(0, NGLOB), :], gq,
                         6 + 3 * j, 6, t).start()

        def process(kref, vref, ncols, cbase, rects, mask_mode, init):
            del ncols
            for r0, nr, c0, nc in rects:
                qi = my * S + r0 + lax.broadcasted_iota(
                    jnp.int32, (nr, nc), 0)
                ki = cbase + c0 + lax.broadcasted_iota(
                    jnp.int32, (nr, nc), 1)
                if mask_mode == "full":
                    mask = ((jnp.abs(qi - ki) <= WINDOW)
                            | (ki < NGLOB) | (qi < NGLOB))
                elif mask_mode == "window32":
                    mask = (jnp.abs(qi - ki) <= WINDOW) & (qi >= NGLOB)
                else:
                    mask = None
                bias = (None if mask is None
                        else jnp.where(mask, 0.0, -1e9).astype(jnp.float32))
                ones_col = jnp.ones((nc, 1), jnp.bfloat16)
                for b in range(B):
                    kb = kref[b, c0:c0 + nc, :]
                    vb = vref[b, c0:c0 + nc, :]
                    for h in range(HQ):
                        sl = slice(h * DH, (h + 1) * DH)
                        s = lax.dot_general(
                            q_ref[b, r0:r0 + nr, sl], kb[:, sl],
                            (((1,), (1,)), ((), ())),
                            preferred_element_type=jnp.float32,
                        )
                        if bias is not None:
                            s = s + bias
                        w = jnp.exp(s).astype(jnp.bfloat16)
                        lsum = jnp.dot(w, ones_col,
                                       preferred_element_type=jnp.float32)
                        pv = jnp.dot(w, vb[:, sl],
                                     preferred_element_type=jnp.float32)
                        if init:
                            l_ref[b, r0:r0 + nr, h:h + 1] = lsum
                            acc_ref[b, r0:r0 + nr, sl] = pv
                        else:
                            l_ref[b, r0:r0 + nr, h:h + 1] += lsum
                            acc_ref[b, r0:r0 + nr, sl] += pv

        if _DO_COMPUTE:
            process(kown, vown, S, my * S,
                    [(0, 256, 0, 384), (256, 256, 128, 384)], "full", True)
            process(kown, vown, S, my * S,
                    [(256, 256, 0, NGLOB), (0, NGLOB, 384, WINDOW)],
                    "full", False)

        grecv_k = rdma(kown.at[:, pl.ds(0, NGLOB), :], gk, 4, 4, 0)
        grecv_v = rdma(vown.at[:, pl.ds(0, NGLOB), :], gv, 4, 5, 0)
        grecv_q = rdma(q_ref.at[:, pl.ds(0, NGLOB), :], gq, 4, 6, 0)
        psend_a = rdma(pacc_s, pacc_r.at[my - 1], 4, 5 + 2 * my, 0)
        psend_l = rdma(pl_s, pl_r.at[my - 1], 5, 6 + 2 * my, 0)

        @pl.when(my != 0)
        def _():
            if _DO_COMM:
                grecv_q.wait_recv()
            if _DO_COMPUTE:
                ones_col = jnp.ones((S, 1), jnp.bfloat16)
                for b in range(B):
                    for h in range(HQ):
                        sl = slice(h * DH, (h + 1) * DH)
                        s = lax.dot_general(
                            gq[b, :, sl], kown[b, :, sl],
                            (((1,), (1,)), ((), ())),
                            preferred_element_type=jnp.float32,
                        )
                        w = jnp.exp(s).astype(jnp.bfloat16)
                        pl_s[b, :, h:h + 1] = jnp.dot(
                            w, ones_col, preferred_element_type=jnp.float32)
                        pacc_s[b, :, sl] = jnp.dot(
                            w, vown[b, :, sl],
                            preferred_element_type=jnp.float32)
            if _DO_COMM:
                psend_a.start()
                psend_l.start()

        if _DO_COMM:
            hkr.wait_recv()
            hvr.wait_recv()
            hkl.wait_recv()
            hvl.wait_recv()
        if _DO_COMPUTE:
            process(hlk, hlv, WINDOW, left * S + (S - WINDOW),
                    [(0, WINDOW, 0, WINDOW)], "window32", False)
            process(hrk, hrv, WINDOW, right * S,
                    [(S - WINDOW, WINDOW, 0, WINDOW)], "window32", False)

        @pl.when(my != 0)
        def _():
            if _DO_COMM:
                grecv_k.wait_recv()
                grecv_v.wait_recv()
            if _DO_COMPUTE:
                process(gk, gv, NGLOB, 0, [(0, S, 0, NGLOB)], "none", False)

        @pl.when(my == 0)
        def _():
            if _DO_COMM:
                for j in range(3):
                    rdma(pacc_s, pacc_r.at[j], 4, 7 + 2 * j, 0).wait_recv()
                    rdma(pl_s, pl_r.at[j], 5, 8 + 2 * j, 0).wait_recv()
            if _DO_COMPUTE:
                for j in range(3):
                    for b in range(B):
                        acc_ref[b, 0:NGLOB, :] += pacc_r[j, b]
                        l_ref[b, 0:NGLOB, :] += pl_r[j, b]

        if _DO_COMM:
            hkr.wait_send()
            hvr.wait_send()
            hkl.wait_send()
            hvl.wait_send()

            @pl.when(my == 0)
            def _():
                for j in range(3):
                    rdma(kown.at[:, pl.ds(0, NGLOB), :], gk,
                         4 + 3 * j, 4, 1).wait_send()
                    rdma(vown.at[:, pl.ds(0, NGLOB), :], gv,
                         5 + 3 * j, 5, 1).wait_send()
                    rdma(q_ref.at[:, pl.ds(0, NGLOB), :], gq,
                         6 + 3 * j, 6, 1).wait_send()

            @pl.when(my != 0)
            def _():
                psend_a.wait_send()
                psend_l.wait_send()

        wo = wo_ref[...].astype(jnp.bfloat16)
        for b in range(B):
            linv = 1.0 / l_ref[b]
            for h in range(HQ):
                sl = slice(h * DH, (h + 1) * DH)
                q_ref[b, :, sl] = (
                    acc_ref[b, :, sl] * linv[:, h:h + 1]
                ).astype(jnp.bfloat16)
        for b in range(B):
            out_ref[b] = jnp.dot(q_ref[b], wo,
                                 preferred_element_type=jnp.float32)

        @functools.partial(pl.run_scoped,
                           second_barrier=pltpu.SemaphoreType.REGULAR)
        def _(second_barrier):
            for d in range(N_DEV):
                pl.semaphore_signal(
                    second_barrier, inc=1,
                    device_id=(jnp.int32(d),),
                    device_id_type=pl.DeviceIdType.MESH,
                )
            pl.semaphore_wait(second_barrier, N_DEV)

    return pl.pallas_call(
        body,
        out_shape=jax.ShapeDtypeStruct((B, S, E), jnp.float32),
        in_specs=[
            pl.BlockSpec(memory_space=pltpu.VMEM),
            pl.BlockSpec(memory_space=pltpu.VMEM),
            pl.BlockSpec(memory_space=pl.ANY),
            pl.BlockSpec(memory_space=pl.ANY),
            pl.BlockSpec(memory_space=pltpu.VMEM),
        ],
        out_specs=pl.BlockSpec(memory_space=pltpu.VMEM),
        scratch_shapes=[
            pltpu.VMEM((B, S, HD), jnp.float32),
            pltpu.VMEM((B, S, HD), jnp.float32),
            pltpu.VMEM((B, S, HD), jnp.bfloat16),
            pltpu.VMEM((B, S, HD), jnp.bfloat16),
            pltpu.VMEM((B, S, HD), jnp.bfloat16),
            pltpu.VMEM((B, S, HD), jnp.float32),
            pltpu.VMEM((B, S, HQ), jnp.float32),
            pltpu.VMEM((B, WINDOW, HD), jnp.bfloat16),
            pltpu.VMEM((B, WINDOW, HD), jnp.bfloat16),
            pltpu.VMEM((B, WINDOW, HD), jnp.bfloat16),
            pltpu.VMEM((B, WINDOW, HD), jnp.bfloat16),
            pltpu.VMEM((B, NGLOB, HD), jnp.bfloat16),
            pltpu.VMEM((B, NGLOB, HD), jnp.bfloat16),
            pltpu.VMEM((B, NGLOB, HD), jnp.bfloat16),
            pltpu.VMEM((B, NGLOB, HD), jnp.float32),
            pltpu.VMEM((B, NGLOB, HQ), jnp.float32),
            pltpu.VMEM((3, B, NGLOB, HD), jnp.float32),
            pltpu.VMEM((3, B, NGLOB, HQ), jnp.float32),
            pltpu.SemaphoreType.DMA((2,)),
            pltpu.SemaphoreType.DMA((13,)),
            pltpu.SemaphoreType.DMA((13,)),
        ],
        compiler_params=pltpu.CompilerParams(collective_id=0),
    )(x, Wq, k2, v2, Wo)


def kernel(x, Wq, K_ext, V_ext, Wo):
    k2 = K_ext.reshape(B, S, HD)
    v2 = V_ext.reshape(B, S, HD)
    return _fused(x, Wq, k2, v2, Wo)


# device time: 35033 ns/iter; 1.1495x vs baseline; 1.0069x over previous
import functools
import os

import jax
import jax.numpy as jnp
from jax import lax
from jax.experimental import pallas as pl
from jax.experimental.pallas import tpu as pltpu

_MODE = os.environ.get("SPARSE_KERNEL_MODE", "")
_DO_COMM = _MODE != "compute_only"
_DO_COMPUTE = _MODE != "comm_only"

N_DEV = 4
B = 2
S = 512
HQ = 8
DH = 64
HD = HQ * DH
E = 768
WINDOW = 128
NGLOB = 32
SCALE = 0.125


def _fused(x, Wq, k2, v2, Wo):

    def body(x_ref, wq_ref, k2_ref, v2_ref, wo_ref, out_ref,
             kf32, vf32, kown, vown, q_ref, acc_ref, l_ref,
             hlk, hlv, hrk, hrv, gk, gv, gq,
             pacc_s, pl_s, pacc_r, pl_r,
             copy_sems, send_sems, recv_sems):
        my = lax.axis_index("i")
        left = lax.rem(my + N_DEV - 1, N_DEV)
        right = lax.rem(my + 1, N_DEV)

        kcopy = pltpu.make_async_copy(k2_ref, kf32, copy_sems.at[0])
        vcopy = pltpu.make_async_copy(v2_ref, vf32, copy_sems.at[1])
        kcopy.start()
        vcopy.start()

        barrier_sem = pltpu.get_barrier_semaphore()
        for d in range(N_DEV):
            pl.semaphore_signal(
                barrier_sem, inc=1,
                device_id=(jnp.int32(d),),
                device_id_type=pl.DeviceIdType.MESH,
            )

        wq = (wq_ref[...] * SCALE).astype(jnp.bfloat16)
        for b in range(B):
            q_ref[b] = jnp.dot(
                x_ref[b].astype(jnp.bfloat16), wq,
                preferred_element_type=jnp.float32,
            ).astype(jnp.bfloat16)

        pl.semaphore_wait(barrier_sem, N_DEV)
        kcopy.wait()
        vcopy.wait()

        for b in range(B):
            kown[b] = kf32[b].astype(jnp.bfloat16)
            vown[b] = vf32[b].astype(jnp.bfloat16)

        def rdma(src, dst, si, ri, dev):
            return pltpu.make_async_remote_copy(
                src_ref=src, dst_ref=dst,
                send_sem=send_sems.at[si], recv_sem=recv_sems.at[ri],
                device_id=(dev,), device_id_type=pl.DeviceIdType.MESH,
            )

        hkr = rdma(kown.at[:, pl.ds(S - WINDOW, WINDOW), :], hlk, 0, 0, right)
        hvr = rdma(vown.at[:, pl.ds(S - WINDOW, WINDOW), :], hlv, 1, 1, right)
        hkl = rdma(kown.at[:, pl.ds(0, WINDOW), :], hrk, 2, 2, left)
        hvl = rdma(vown.at[:, pl.ds(0, WINDOW), :], hrv, 3, 3, left)
        if _DO_COMM:
            hkr.start()
            hvr.start()
            hkl.start()
            hvl.start()

            @pl.when(my == 0)
            def _():
                for j, t in enumerate((1, 2, 3)):
                    rdma(kown.at[:, pl.ds(0, NGLOB), :], gk,
                         4 + 3 * j, 4, t).start()
                    rdma(vown.at[:, pl.ds(0, NGLOB), :], gv,
                         5 + 3 * j, 5, t).start()

        if _DO_COMM:
            @pl.when(my == 0)
            def _():
                for j, t in enumerate((1, 2, 3)):
                    rdma(q_ref.at[:, pl.ds(0, NGLOB), :], gq,
                         6 + 3 * j, 6, t).start()

        def process(kref, vref, ncols, cbase, rects, mask_mode, init):
            del ncols
            for r0, nr, c0, nc in rects:
                qi = my * S + r0 + lax.broadcasted_iota(
                    jnp.int32, (nr, nc), 0)
                ki = cbase + c0 + lax.broadcasted_iota(
                    jnp.int32, (nr, nc), 1)
                if mask_mode == "full":
                    mask = ((jnp.abs(qi - ki) <= WINDOW)
                            | (ki < NGLOB) | (qi < NGLOB))
                elif mask_mode == "window32":
                    mask = (jnp.abs(qi - ki) <= WINDOW) & (qi >= NGLOB)
                else:
                    mask = None
                bias = (None if mask is None
                        else jnp.where(mask, 0.0, -1e9).astype(jnp.float32))
                ones_col = jnp.ones((nc, 1), jnp.bfloat16)
                for b in range(B):
                    kb = kref[b, c0:c0 + nc, :]
                    vb = vref[b, c0:c0 + nc, :]
                    for h in range(HQ):
                        sl = slice(h * DH, (h + 1) * DH)
                        s = lax.dot_general(
                            q_ref[b, r0:r0 + nr, sl], kb[:, sl],
                            (((1,), (1,)), ((), ())),
                            preferred_element_type=jnp.float32,
                        )
                        if bias is not None:
                            s = s + bias
                        w = jnp.exp(s).astype(jnp.bfloat16)
                        lsum = jnp.dot(w, ones_col,
                                       preferred_element_type=jnp.float32)
                        pv = jnp.dot(w, vb[:, sl],
                                     preferred_element_type=jnp.float32)
                        if init:
                            l_ref[b, r0:r0 + nr, h:h + 1] = lsum
                            acc_ref[b, r0:r0 + nr, sl] = pv
                        else:
                            l_ref[b, r0:r0 + nr, h:h + 1] += lsum
                            acc_ref[b, r0:r0 + nr, sl] += pv

        if _DO_COMPUTE:
            process(kown, vown, S, my * S,
                    [(0, 256, 0, 384), (256, 256, 128, 384)], "full", True)
            process(kown, vown, S, my * S,
                    [(256, 256, 0, NGLOB), (0, NGLOB, 384, WINDOW)],
                    "full", False)

        grecv_k = rdma(kown.at[:, pl.ds(0, NGLOB), :], gk, 4, 4, 0)
        grecv_v = rdma(vown.at[:, pl.ds(0, NGLOB), :], gv, 4, 5, 0)
        grecv_q = rdma(q_ref.at[:, pl.ds(0, NGLOB), :], gq, 4, 6, 0)
        psend_a = rdma(pacc_s, pacc_r.at[my - 1], 4, 5 + 2 * my, 0)
        psend_l = rdma(pl_s, pl_r.at[my - 1], 5, 6 + 2 * my, 0)

        @pl.when(my != 0)
        def _():
            if _DO_COMM:
                grecv_q.wait_recv()
            if _DO_COMPUTE:
                ones_col = jnp.ones((S, 1), jnp.bfloat16)
                for b in range(B):
                    for h in range(HQ):
                        sl = slice(h * DH, (h + 1) * DH)
                        s = lax.dot_general(
                            gq[b, :, sl], kown[b, :, sl],
                            (((1,), (1,)), ((), ())),
                            preferred_element_type=jnp.float32,
                        )
                        w = jnp.exp(s).astype(jnp.bfloat16)
                        pl_s[b, :, h:h + 1] = jnp.dot(
                            w, ones_col, preferred_element_type=jnp.float32)
                        pacc_s[b, :, sl] = jnp.dot(
                            w, vown[b, :, sl],
                            preferred_element_type=jnp.float32)
            if _DO_COMM:
                psend_a.start()
                psend_l.start()

        if _DO_COMM:
            hkr.wait_recv()
            hvr.wait_recv()
            hkl.wait_recv()
            hvl.wait_recv()
        if _DO_COMPUTE:
            process(hlk, hlv, WINDOW, left * S + (S - WINDOW),
                    [(0, WINDOW, 0, WINDOW)], "window32", False)
            process(hrk, hrv, WINDOW, right * S,
                    [(S - WINDOW, WINDOW, 0, WINDOW)], "window32", False)

        @pl.when(my != 0)
        def _():
            if _DO_COMM:
                grecv_k.wait_recv()
                grecv_v.wait_recv()
            if _DO_COMPUTE:
                process(gk, gv, NGLOB, 0, [(0, S, 0, NGLOB)], "none", False)

        @pl.when(my == 0)
        def _():
            if _DO_COMM:
                for j in range(3):
                    rdma(pacc_s, pacc_r.at[j], 4, 7 + 2 * j, 0).wait_recv()
                    rdma(pl_s, pl_r.at[j], 5, 8 + 2 * j, 0).wait_recv()
            if _DO_COMPUTE:
                for j in range(3):
                    for b in range(B):
                        acc_ref[b, 0:NGLOB, :] += pacc_r[j, b]
                        l_ref[b, 0:NGLOB, :] += pl_r[j, b]

        if _DO_COMM:
            hkr.wait_send()
            hvr.wait_send()
            hkl.wait_send()
            hvl.wait_send()

            @pl.when(my == 0)
            def _():
                for j in range(3):
                    rdma(kown.at[:, pl.ds(0, NGLOB), :], gk,
                         4 + 3 * j, 4, 1).wait_send()
                    rdma(vown.at[:, pl.ds(0, NGLOB), :], gv,
                         5 + 3 * j, 5, 1).wait_send()
                    rdma(q_ref.at[:, pl.ds(0, NGLOB), :], gq,
                         6 + 3 * j, 6, 1).wait_send()

            @pl.when(my != 0)
            def _():
                psend_a.wait_send()
                psend_l.wait_send()

        wo = wo_ref[...].astype(jnp.bfloat16)
        for b in range(B):
            linv = 1.0 / l_ref[b]
            for h in range(HQ):
                sl = slice(h * DH, (h + 1) * DH)
                q_ref[b, :, sl] = (
                    acc_ref[b, :, sl] * linv[:, h:h + 1]
                ).astype(jnp.bfloat16)
        for b in range(B):
            out_ref[b] = jnp.dot(q_ref[b], wo,
                                 preferred_element_type=jnp.float32
                                 ).astype(jnp.bfloat16)

        @functools.partial(pl.run_scoped,
                           second_barrier=pltpu.SemaphoreType.REGULAR)
        def _(second_barrier):
            for d in range(N_DEV):
                pl.semaphore_signal(
                    second_barrier, inc=1,
                    device_id=(jnp.int32(d),),
                    device_id_type=pl.DeviceIdType.MESH,
                )
            pl.semaphore_wait(second_barrier, N_DEV)

    return pl.pallas_call(
        body,
        out_shape=jax.ShapeDtypeStruct((B, S, E), jnp.bfloat16),
        in_specs=[
            pl.BlockSpec(memory_space=pltpu.VMEM),
            pl.BlockSpec(memory_space=pltpu.VMEM),
            pl.BlockSpec(memory_space=pl.ANY),
            pl.BlockSpec(memory_space=pl.ANY),
            pl.BlockSpec(memory_space=pltpu.VMEM),
        ],
        out_specs=pl.BlockSpec(memory_space=pltpu.VMEM),
        scratch_shapes=[
            pltpu.VMEM((B, S, HD), jnp.float32),
            pltpu.VMEM((B, S, HD), jnp.float32),
            pltpu.VMEM((B, S, HD), jnp.bfloat16),
            pltpu.VMEM((B, S, HD), jnp.bfloat16),
            pltpu.VMEM((B, S, HD), jnp.bfloat16),
            pltpu.VMEM((B, S, HD), jnp.float32),
            pltpu.VMEM((B, S, HQ), jnp.float32),
            pltpu.VMEM((B, WINDOW, HD), jnp.bfloat16),
            pltpu.VMEM((B, WINDOW, HD), jnp.bfloat16),
            pltpu.VMEM((B, WINDOW, HD), jnp.bfloat16),
            pltpu.VMEM((B, WINDOW, HD), jnp.bfloat16),
            pltpu.VMEM((B, NGLOB, HD), jnp.bfloat16),
            pltpu.VMEM((B, NGLOB, HD), jnp.bfloat16),
            pltpu.VMEM((B, NGLOB, HD), jnp.bfloat16),
            pltpu.VMEM((B, NGLOB, HD), jnp.float32),
            pltpu.VMEM((B, NGLOB, HQ), jnp.float32),
            pltpu.VMEM((3, B, NGLOB, HD), jnp.float32),
            pltpu.VMEM((3, B, NGLOB, HQ), jnp.float32),
            pltpu.SemaphoreType.DMA((2,)),
            pltpu.SemaphoreType.DMA((13,)),
            pltpu.SemaphoreType.DMA((13,)),
        ],
        compiler_params=pltpu.CompilerParams(collective_id=0),
    )(x, Wq, k2, v2, Wo)


def kernel(x, Wq, K_ext, V_ext, Wo):
    k2 = K_ext.reshape(B, S, HD)
    v2 = V_ext.reshape(B, S, HD)
    return _fused(x, Wq, k2, v2, Wo)
